# baseline (speedup 1.0000x reference)
import functools

import jax
import jax.numpy as jnp
from jax import lax
from jax.experimental import pallas as pl
from jax.experimental.pallas import tpu as pltpu
from jax.experimental.pallas import tpu_sc as plsc

B = 16384
TB = 2048
NW = 32
RPW = B // NW
NCH = RPW // 128


def _prep_body(se_ref, ge_ref, me_ref, Wsub_ref, bsub_ref, Wgr_ref, bgr_ref,
               W1_ref, b1_ref, ts_o, tg_o, tmb_o, wsub1_o, wgr1_o):
    f32 = jnp.float32
    W1 = W1_ref[...]
    ts_o[...] = jnp.dot(se_ref[...], W1[0:32, :], preferred_element_type=f32)
    tg_o[...] = jnp.dot(ge_ref[...], W1[32:64, :], preferred_element_type=f32)
    bias = (b1_ref[...]
            + jnp.dot(bsub_ref[...], W1[96:128, :], preferred_element_type=f32)
            + jnp.dot(bgr_ref[...], W1[128:160, :], preferred_element_type=f32))
    tmb_o[...] = jnp.dot(me_ref[...], W1[64:96, :], preferred_element_type=f32) + bias
    wsub1_o[...] = jnp.dot(Wsub_ref[...], W1[96:128, :], preferred_element_type=f32)
    wgr1_o[...] = jnp.dot(Wgr_ref[...], W1[128:160, :], preferred_element_type=f32)


def _prep(school_emb, goal_emb, method_emb, W_sub, b_sub, W_gr, b_gr, W1, b1):
    shp = lambda r: jax.ShapeDtypeStruct((r, 128), jnp.float32)
    return pl.pallas_call(
        _prep_body,
        out_shape=(shp(102), shp(22), shp(12), shp(15), shp(12)),
    )(school_emb, goal_emb, method_emb, W_sub, b_sub.reshape(1, 32),
      W_gr, b_gr.reshape(1, 32), W1, b1.reshape(1, 128))


def _sc_body(si_hbm, gi_hbm, mi_hbm, ts_hbm, tg_hbm, tmb_hbm, out_hbm,
             idx_s, idx_g, idx_m, lin_v, buf, acc, sem):
    cid = lax.axis_index("c")
    sid = lax.axis_index("s")
    wid = cid * 16 + sid
    lbase = sid * RPW
    gbase = wid * RPW
    grow = wid * NCH

    pltpu.sync_copy(si_hbm.at[pl.ds(grow, NCH)], idx_s)
    pltpu.sync_copy(gi_hbm.at[pl.ds(grow, NCH)], idx_g)
    pltpu.sync_copy(mi_hbm.at[pl.ds(grow, NCH)], idx_m)

    for j in range(NCH):
        for k in range(8):
            lin_v[j, pl.ds(k * 16, 16)] = (
                lax.iota(jnp.int32, 16) + (lbase + j * 128 + k * 16))

    for j in range(NCH):
        pltpu.async_copy(ts_hbm.at[idx_s.at[j]], buf, sem).wait()
        pltpu.sync_copy(buf, acc.at[pl.ds(lbase + j * 128, 128)])
    for j in range(NCH):
        pltpu.async_copy(tg_hbm.at[idx_g.at[j]], buf, sem).wait()
        pltpu.sync_copy(buf, acc.at[lin_v.at[j]], add=True)
    for j in range(NCH):
        pltpu.async_copy(tmb_hbm.at[idx_m.at[j]], buf, sem).wait()
        pltpu.sync_copy(buf, acc.at[lin_v.at[j]], add=True)

    pltpu.sync_copy(acc.at[pl.ds(lbase, RPW)], out_hbm.at[pl.ds(gbase, RPW)])


def _sc_gather(si2, gi2, mi2, ts, tg, tmb):
    mesh = plsc.VectorSubcoreMesh(core_axis_name="c", subcore_axis_name="s")
    k = functools.partial(
        pl.kernel,
        mesh=mesh,
        out_type=jax.ShapeDtypeStruct((B, 128), jnp.float32),
        scratch_types=[
            pltpu.VMEM((NCH, 128), jnp.int32),
            pltpu.VMEM((NCH, 128), jnp.int32),
            pltpu.VMEM((NCH, 128), jnp.int32),
            pltpu.VMEM((NCH, 128), jnp.int32),
            pltpu.VMEM((128, 128), jnp.float32),
            pltpu.VMEM_SHARED((B // 2, 128), jnp.float32),
            pltpu.SemaphoreType.DMA,
        ],
    )(_sc_body)
    return k(si2, gi2, mi2, ts, tg, tmb)


def _tail_body(e_ref, subM_ref, grM_ref, wsub1_ref, wgr1_ref,
               W2_ref, b2_ref, W3_ref, b3_ref, out_ref):
    f32 = jnp.float32
    h1 = (e_ref[...]
          + jnp.dot(subM_ref[...], wsub1_ref[...], preferred_element_type=f32)
          + jnp.dot(grM_ref[...], wgr1_ref[...], preferred_element_type=f32))
    h1 = jnp.maximum(h1, 0.0)
    h2 = jnp.maximum(jnp.dot(h1, W2_ref[...], preferred_element_type=f32) + b2_ref[...], 0.0)
    out_ref[...] = jnp.dot(h2, W3_ref[...], preferred_element_type=f32) + b3_ref[...]


def _tail(E, subM, grM, wsub1, wgr1, W2, b2, W3, b3):
    nb = B // TB

    def batch_spec(w):
        return pl.BlockSpec((TB, w), lambda i: (i, 0))

    def full_spec(shape):
        return pl.BlockSpec(shape, lambda i: (0,) * len(shape))

    return pl.pallas_call(
        _tail_body,
        grid=(nb,),
        in_specs=[
            batch_spec(128), batch_spec(15), batch_spec(12),
            full_spec((15, 128)), full_spec((12, 128)),
            full_spec((128, 64)), full_spec((1, 64)),
            full_spec((64, 32)), full_spec((1, 32)),
        ],
        out_specs=pl.BlockSpec((TB, 32), lambda i: (i, 0)),
        out_shape=jax.ShapeDtypeStruct((B, 32), jnp.float32),
    )(E, subM, grM, wsub1, wgr1, W2, b2.reshape(1, 64), W3, b3.reshape(1, 32))


def kernel(school_idx, goal_idx, method_idx, subject_multi_hot, grade_multi_hot,
           school_emb, goal_emb, method_emb, W_sub, b_sub, W_gr, b_gr,
           W1, b1, W2, b2, W3, b3):
    ts, tg, tmb, wsub1, wgr1 = _prep(
        school_emb, goal_emb, method_emb, W_sub, b_sub, W_gr, b_gr, W1, b1)
    si2 = school_idx.astype(jnp.int32).reshape(B // 128, 128)
    gi2 = goal_idx.astype(jnp.int32).reshape(B // 128, 128)
    mi2 = method_idx.astype(jnp.int32).reshape(B // 128, 128)
    E = _sc_gather(si2, gi2, mi2, ts, tg, tmb)
    return _tail(E, subject_multi_hot, grade_multi_hot, wsub1, wgr1, W2, b2, W3, b3)


# speedup vs baseline: 1.0466x; 1.0466x over previous
import functools

import jax
import jax.numpy as jnp
from jax import lax
from jax.experimental import pallas as pl
from jax.experimental.pallas import tpu as pltpu
from jax.experimental.pallas import tpu_sc as plsc

B = 16384
TB = 2048
NW = 32
RPW = B // NW
NCH = RPW // 128


def _prep_body(se_ref, ge_ref, me_ref, Wsub_ref, bsub_ref, Wgr_ref, bgr_ref,
               W1_ref, b1_ref, ts_o, tg_o, tmb_o, wsub1_o, wgr1_o):
    f32 = jnp.float32
    W1 = W1_ref[...]
    ts_o[...] = jnp.dot(se_ref[...], W1[0:32, :], preferred_element_type=f32)
    tg_o[...] = jnp.dot(ge_ref[...], W1[32:64, :], preferred_element_type=f32)
    bias = (b1_ref[...]
            + jnp.dot(bsub_ref[...], W1[96:128, :], preferred_element_type=f32)
            + jnp.dot(bgr_ref[...], W1[128:160, :], preferred_element_type=f32))
    tmb_o[...] = jnp.dot(me_ref[...], W1[64:96, :], preferred_element_type=f32) + bias
    wsub1_o[...] = jnp.dot(Wsub_ref[...], W1[96:128, :], preferred_element_type=f32)
    wgr1_o[...] = jnp.dot(Wgr_ref[...], W1[128:160, :], preferred_element_type=f32)


def _prep(school_emb, goal_emb, method_emb, W_sub, b_sub, W_gr, b_gr, W1, b1):
    shp = lambda r: jax.ShapeDtypeStruct((r, 128), jnp.float32)
    return pl.pallas_call(
        _prep_body,
        out_shape=(shp(102), shp(22), shp(12), shp(15), shp(12)),
    )(school_emb, goal_emb, method_emb, W_sub, b_sub.reshape(1, 32),
      W_gr, b_gr.reshape(1, 32), W1, b1.reshape(1, 128))


NBUF = 3
LOOKAHEAD = 2


def _sc_body(si_hbm, gi_hbm, mi_hbm, ts_hbm, tg_hbm, tmb_hbm, out_hbm,
             idx_s, idx_g, idx_m, lin_v, bufs, acc, semg, semc):
    cid = lax.axis_index("c")
    sid = lax.axis_index("s")
    wid = cid * 16 + sid
    lbase = sid * RPW
    gbase = wid * RPW
    grow = wid * NCH

    pltpu.sync_copy(si_hbm.at[pl.ds(grow, NCH)], idx_s)
    pltpu.sync_copy(gi_hbm.at[pl.ds(grow, NCH)], idx_g)
    pltpu.sync_copy(mi_hbm.at[pl.ds(grow, NCH)], idx_m)

    for j in range(NCH):
        for k in range(8):
            lin_v[j, pl.ds(k * 16, 16)] = (
                lax.iota(jnp.int32, 16) + (lbase + j * 128 + k * 16))

    chunks = ([(ts_hbm, idx_s, j, 'w') for j in range(NCH)]
              + [(tg_hbm, idx_g, j, 'a') for j in range(NCH)]
              + [(tmb_hbm, idx_m, j, 'a') for j in range(NCH)])
    n = len(chunks)
    gh = [None] * n
    ch = [None] * n

    def wait_ch(u):
        if ch[u] is not None:
            ch[u].wait()
            ch[u] = None

    for t in range(n + LOOKAHEAD):
        if t < n:
            tbl, idx, j, _ = chunks[t]
            b = t % NBUF
            if t >= NBUF:
                wait_ch(t - NBUF)
            gh[t] = pltpu.async_copy(tbl.at[idx.at[j]], bufs.at[b], semg.at[b])
        u = t - LOOKAHEAD
        if 0 <= u < n:
            if u == NCH:
                for v in range(NCH):
                    wait_ch(v)
            _, _, j, mode = chunks[u]
            b = u % NBUF
            gh[u].wait()
            if mode == 'w':
                ch[u] = pltpu.async_copy(
                    bufs.at[b], acc.at[pl.ds(lbase + j * 128, 128)], semc.at[b])
            else:
                ch[u] = pltpu.async_copy(
                    bufs.at[b], acc.at[lin_v.at[j]], semc.at[b], add=True)

    for u in range(n):
        wait_ch(u)
    pltpu.sync_copy(acc.at[pl.ds(lbase, RPW)], out_hbm.at[pl.ds(gbase, RPW)])


def _sc_gather(si2, gi2, mi2, ts, tg, tmb):
    mesh = plsc.VectorSubcoreMesh(core_axis_name="c", subcore_axis_name="s")
    k = functools.partial(
        pl.kernel,
        mesh=mesh,
        out_type=jax.ShapeDtypeStruct((B, 128), jnp.float32),
        scratch_types=[
            pltpu.VMEM((NCH, 128), jnp.int32),
            pltpu.VMEM((NCH, 128), jnp.int32),
            pltpu.VMEM((NCH, 128), jnp.int32),
            pltpu.VMEM((NCH, 128), jnp.int32),
            pltpu.VMEM((NBUF, 128, 128), jnp.float32),
            pltpu.VMEM_SHARED((B // 2, 128), jnp.float32),
            pltpu.SemaphoreType.DMA((NBUF,)),
            pltpu.SemaphoreType.DMA((NBUF,)),
        ],
    )(_sc_body)
    return k(si2, gi2, mi2, ts, tg, tmb)


def _tail_body(e_ref, subM_ref, grM_ref, wsub1_ref, wgr1_ref,
               W2_ref, b2_ref, W3_ref, b3_ref, out_ref):
    f32 = jnp.float32
    h1 = (e_ref[...]
          + jnp.dot(subM_ref[...], wsub1_ref[...], preferred_element_type=f32)
          + jnp.dot(grM_ref[...], wgr1_ref[...], preferred_element_type=f32))
    h1 = jnp.maximum(h1, 0.0)
    h2 = jnp.maximum(jnp.dot(h1, W2_ref[...], preferred_element_type=f32) + b2_ref[...], 0.0)
    out_ref[...] = jnp.dot(h2, W3_ref[...], preferred_element_type=f32) + b3_ref[...]


def _tail(E, subM, grM, wsub1, wgr1, W2, b2, W3, b3):
    nb = B // TB

    def batch_spec(w):
        return pl.BlockSpec((TB, w), lambda i: (i, 0))

    def full_spec(shape):
        return pl.BlockSpec(shape, lambda i: (0,) * len(shape))

    return pl.pallas_call(
        _tail_body,
        grid=(nb,),
        in_specs=[
            batch_spec(128), batch_spec(15), batch_spec(12),
            full_spec((15, 128)), full_spec((12, 128)),
            full_spec((128, 64)), full_spec((1, 64)),
            full_spec((64, 32)), full_spec((1, 32)),
        ],
        out_specs=pl.BlockSpec((TB, 32), lambda i: (i, 0)),
        out_shape=jax.ShapeDtypeStruct((B, 32), jnp.float32),
    )(E, subM, grM, wsub1, wgr1, W2, b2.reshape(1, 64), W3, b3.reshape(1, 32))


def kernel(school_idx, goal_idx, method_idx, subject_multi_hot, grade_multi_hot,
           school_emb, goal_emb, method_emb, W_sub, b_sub, W_gr, b_gr,
           W1, b1, W2, b2, W3, b3):
    ts, tg, tmb, wsub1, wgr1 = _prep(
        school_emb, goal_emb, method_emb, W_sub, b_sub, W_gr, b_gr, W1, b1)
    si2 = school_idx.astype(jnp.int32).reshape(B // 128, 128)
    gi2 = goal_idx.astype(jnp.int32).reshape(B // 128, 128)
    mi2 = method_idx.astype(jnp.int32).reshape(B // 128, 128)
    E = _sc_gather(si2, gi2, mi2, ts, tg, tmb)
    return _tail(E, subject_multi_hot, grade_multi_hot, wsub1, wgr1, W2, b2, W3, b3)


# speedup vs baseline: 1.1470x; 1.0959x over previous
import functools

import jax
import jax.numpy as jnp
from jax import lax
from jax.experimental import pallas as pl
from jax.experimental.pallas import tpu as pltpu
from jax.experimental.pallas import tpu_sc as plsc

B = 16384
TB = 2048
NW = 32
RPW = B // NW
NCH = RPW // 128


def _prep_body(se_ref, ge_ref, me_ref, Wsub_ref, bsub_ref, Wgr_ref, bgr_ref,
               W1_ref, b1_ref, ts_o, tg_o, tmb_o, wsub1_o, wgr1_o):
    f32 = jnp.float32
    W1 = W1_ref[...]
    ts_o[...] = jnp.dot(se_ref[...], W1[0:32, :], preferred_element_type=f32)
    tg_o[...] = jnp.dot(ge_ref[...], W1[32:64, :], preferred_element_type=f32)
    bias = (b1_ref[...]
            + jnp.dot(bsub_ref[...], W1[96:128, :], preferred_element_type=f32)
            + jnp.dot(bgr_ref[...], W1[128:160, :], preferred_element_type=f32))
    tmb_o[...] = jnp.dot(me_ref[...], W1[64:96, :], preferred_element_type=f32) + bias
    wsub1_o[...] = jnp.dot(Wsub_ref[...], W1[96:128, :], preferred_element_type=f32)
    wgr1_o[...] = jnp.dot(Wgr_ref[...], W1[128:160, :], preferred_element_type=f32)


def _prep(school_emb, goal_emb, method_emb, W_sub, b_sub, W_gr, b_gr, W1, b1):
    shp = lambda r: jax.ShapeDtypeStruct((r, 128), jnp.float32)
    return pl.pallas_call(
        _prep_body,
        out_shape=(shp(102), shp(22), shp(12), shp(15), shp(12)),
    )(school_emb, goal_emb, method_emb, W_sub, b_sub.reshape(1, 32),
      W_gr, b_gr.reshape(1, 32), W1, b1.reshape(1, 128))


CH = 128
NSET = 2


def _sc_body(si_hbm, gi_hbm, mi_hbm, ts_hbm, tg_hbm, tmb_hbm, out_hbm,
             idx_s, idx_g, idx_m, bufs, gsem, wsem):
    cid = lax.axis_index("c")
    sid = lax.axis_index("s")
    wid = cid * 16 + sid
    gbase = wid * RPW
    grow = wid * NCH

    pltpu.sync_copy(si_hbm.at[pl.ds(grow, NCH)], idx_s)
    pltpu.sync_copy(gi_hbm.at[pl.ds(grow, NCH)], idx_g)
    pltpu.sync_copy(mi_hbm.at[pl.ds(grow, NCH)], idx_m)

    def addpass(d):
        def row_body(r, carry):
            for k in range(8):
                sl = pl.ds(k * 16, 16)
                bufs[d, 2, r, sl] = (bufs[d, 2, r, sl]
                                     + bufs[d, 0, r, sl] + bufs[d, 1, r, sl])
            return carry
        lax.fori_loop(0, CH, row_body, 0)

    gh = [None] * NCH
    wh = [None] * NCH
    for p in range(NCH + NSET):
        q = p - NSET
        if 0 <= q < NCH:
            d = q % NSET
            for h in gh[q]:
                h.wait()
            addpass(d)
            wh[q] = pltpu.async_copy(
                bufs.at[d, 2], out_hbm.at[pl.ds(gbase + q * CH, CH)],
                wsem.at[d])
        if p < NCH:
            d = p % NSET
            g0 = pltpu.async_copy(ts_hbm.at[idx_s.at[p]], bufs.at[d, 0],
                                  gsem.at[d, 0])
            g1 = pltpu.async_copy(tg_hbm.at[idx_g.at[p]], bufs.at[d, 1],
                                  gsem.at[d, 1])
            if p >= NSET:
                wh[p - NSET].wait()
                wh[p - NSET] = None
            g2 = pltpu.async_copy(tmb_hbm.at[idx_m.at[p]], bufs.at[d, 2],
                                  gsem.at[d, 2])
            gh[p] = [g0, g1, g2]

    for q in range(NCH):
        if wh[q] is not None:
            wh[q].wait()


def _sc_gather(si2, gi2, mi2, ts, tg, tmb):
    mesh = plsc.VectorSubcoreMesh(core_axis_name="c", subcore_axis_name="s")
    k = functools.partial(
        pl.kernel,
        mesh=mesh,
        out_type=jax.ShapeDtypeStruct((B, 128), jnp.float32),
        scratch_types=[
            pltpu.VMEM((NCH, 128), jnp.int32),
            pltpu.VMEM((NCH, 128), jnp.int32),
            pltpu.VMEM((NCH, 128), jnp.int32),
            pltpu.VMEM((NSET, 3, CH, 128), jnp.float32),
            pltpu.SemaphoreType.DMA((NSET, 3)),
            pltpu.SemaphoreType.DMA((NSET,)),
        ],
    )(_sc_body)
    return k(si2, gi2, mi2, ts, tg, tmb)


def _tail_body(e_ref, subM_ref, grM_ref, wsub1_ref, wgr1_ref,
               W2_ref, b2_ref, W3_ref, b3_ref, out_ref):
    f32 = jnp.float32
    h1 = (e_ref[...]
          + jnp.dot(subM_ref[...], wsub1_ref[...], preferred_element_type=f32)
          + jnp.dot(grM_ref[...], wgr1_ref[...], preferred_element_type=f32))
    h1 = jnp.maximum(h1, 0.0)
    h2 = jnp.maximum(jnp.dot(h1, W2_ref[...], preferred_element_type=f32) + b2_ref[...], 0.0)
    out_ref[...] = jnp.dot(h2, W3_ref[...], preferred_element_type=f32) + b3_ref[...]


def _tail(E, subM, grM, wsub1, wgr1, W2, b2, W3, b3):
    nb = B // TB

    def batch_spec(w):
        return pl.BlockSpec((TB, w), lambda i: (i, 0))

    def full_spec(shape):
        return pl.BlockSpec(shape, lambda i: (0,) * len(shape))

    return pl.pallas_call(
        _tail_body,
        grid=(nb,),
        in_specs=[
            batch_spec(128), batch_spec(15), batch_spec(12),
            full_spec((15, 128)), full_spec((12, 128)),
            full_spec((128, 64)), full_spec((1, 64)),
            full_spec((64, 32)), full_spec((1, 32)),
        ],
        out_specs=pl.BlockSpec((TB, 32), lambda i: (i, 0)),
        out_shape=jax.ShapeDtypeStruct((B, 32), jnp.float32),
    )(E, subM, grM, wsub1, wgr1, W2, b2.reshape(1, 64), W3, b3.reshape(1, 32))


def kernel(school_idx, goal_idx, method_idx, subject_multi_hot, grade_multi_hot,
           school_emb, goal_emb, method_emb, W_sub, b_sub, W_gr, b_gr,
           W1, b1, W2, b2, W3, b3):
    ts, tg, tmb, wsub1, wgr1 = _prep(
        school_emb, goal_emb, method_emb, W_sub, b_sub, W_gr, b_gr, W1, b1)
    si2 = school_idx.astype(jnp.int32).reshape(B // 128, 128)
    gi2 = goal_idx.astype(jnp.int32).reshape(B // 128, 128)
    mi2 = method_idx.astype(jnp.int32).reshape(B // 128, 128)
    E = _sc_gather(si2, gi2, mi2, ts, tg, tmb)
    return _tail(E, subject_multi_hot, grade_multi_hot, wsub1, wgr1, W2, b2, W3, b3)
